# baseline (speedup 1.0000x reference)
import functools

import jax
import jax.numpy as jnp
from jax import lax
from jax.experimental import pallas as pl
from jax.experimental.pallas import tpu as pltpu
from jax.experimental.pallas import tpu_sc as plsc

B, D, HW = 32, 32, 64 * 64
E = 512
_LOSS_SCALE = 1.25 / (B * HW * D)


def _tc_body(cb_ref, z_ref, idx_ref, loss_ref):
    b = pl.program_id(0)
    zb = z_ref[0].reshape(D, HW)
    cb = cb_ref[...]

    nscores2 = lax.dot_general(-2.0 * cb, zb, (((1,), (0,)), ((), ())),
                               preferred_element_type=jnp.float32)
    e_sq = jnp.sum(cb * cb, axis=1, keepdims=True)
    z_sq = jnp.sum(zb * zb, axis=0, keepdims=True)
    t = z_sq + e_sq
    dist = t + nscores2

    m = jnp.min(dist, axis=0, keepdims=True)
    eidx = lax.broadcasted_iota(jnp.int32, (E, 1), 0).astype(jnp.float32)
    idx_f = jnp.min(jnp.where(dist == m, eidx, float(E)), axis=0)
    idx_ref[0, 0] = idx_f.astype(jnp.int32)

    @pl.when(b == 0)
    def _init():
        loss_ref[0, 0] = 0.0

    loss_ref[0, 0] += jnp.sum(m)


def _tc_stage(codebook, z4, b0, nb):
    return pl.pallas_call(
        _tc_body,
        grid=(nb,),
        in_specs=[
            pl.BlockSpec((E, D), lambda b: (0, 0)),
            pl.BlockSpec((1, D, 64, 64), lambda b: (b0 + b, 0, 0, 0)),
        ],
        out_specs=[
            pl.BlockSpec((1, 1, HW), lambda b: (b, 0, 0)),
            pl.BlockSpec(memory_space=pltpu.SMEM),
        ],
        out_shape=[
            jax.ShapeDtypeStruct((nb, 1, HW), jnp.int32),
            jax.ShapeDtypeStruct((1, 1), jnp.float32),
        ],
    )(codebook, z4)


_CHUNK = 1024
_NCHUNK = HW // _CHUNK


def _sc_gather(cbt, idx3, nb):
    mesh = plsc.VectorSubcoreMesh(core_axis_name="c", subcore_axis_name="s")
    wpb = 32 // nb
    per_w = HW // wpb
    nchunk = per_w // _CHUNK

    @functools.partial(
        pl.kernel,
        out_type=jax.ShapeDtypeStruct((nb, D, HW), jnp.float32),
        mesh=mesh,
        compiler_params=pltpu.CompilerParams(
            use_tc_tiling_on_sc=False, needs_layout_passes=False),
        scratch_types=[
            pltpu.VMEM((D * E,), jnp.float32),
            pltpu.VMEM((per_w,), jnp.int32),
            pltpu.VMEM((D, _CHUNK), jnp.float32),
            pltpu.VMEM((D, _CHUNK), jnp.float32),
            pltpu.SemaphoreType.DMA,
            pltpu.SemaphoreType.DMA,
        ],
    )
    def body(cbt_hbm, idx_hbm, zq_hbm, cbt_v, idx_v, out0, out1, sem0, sem1):
        w = lax.axis_index("s") * 2 + lax.axis_index("c")
        wb = w // wpb
        off = (w % wpb) * per_w
        bufs, sems = (out0, out1), (sem0, sem1)
        pltpu.sync_copy(cbt_hbm, cbt_v)
        pltpu.sync_copy(idx_hbm.at[wb, 0, pl.ds(off, per_w)], idx_v)
        for c in range(nchunk):
            buf, sem = bufs[c % 2], sems[c % 2]
            if c >= 2:
                pltpu.make_async_copy(
                    buf, zq_hbm.at[wb, :, pl.ds(off + (c - 2) * _CHUNK,
                                                _CHUNK)], sem).wait()

            @plsc.parallel_loop(0, _CHUNK // 16, 1, unroll=4)
            def g_body(g, _c=c, _buf=buf):
                iv = idx_v[pl.ds(_c * _CHUNK + g * 16, 16)]
                for d in range(D):
                    _buf[d, pl.ds(g * 16, 16)] = plsc.load_gather(
                        cbt_v, [iv + (d * E)])

            pltpu.async_copy(
                buf, zq_hbm.at[wb, :, pl.ds(off + c * _CHUNK, _CHUNK)], sem)
        for c in (nchunk - 2, nchunk - 1):
            pltpu.make_async_copy(
                bufs[c % 2],
                zq_hbm.at[wb, :, pl.ds(off + c * _CHUNK, _CHUNK)],
                sems[c % 2]).wait()

    return body(cbt, idx3)


def kernel(z_e, codebook):
    nb = B // 2
    idx_a, loss_a = _tc_stage(codebook, z_e, 0, nb)
    idx_b, loss_b = _tc_stage(codebook, z_e, nb, nb)
    cbt = codebook.T.reshape(-1)
    zq_a = _sc_gather(cbt, idx_a, nb)
    zq_b = _sc_gather(cbt, idx_b, nb)
    zq3 = jnp.concatenate((zq_a, zq_b), axis=0)
    z_q = zq3.reshape(B, D, 64, 64)
    indices = jnp.concatenate((idx_a, idx_b), axis=0).reshape(B, 64, 64)
    loss = (loss_a[0, 0] + loss_b[0, 0]) * _LOSS_SCALE
    return (z_q, indices, loss)


# speedup vs baseline: 1.1181x; 1.1181x over previous
import functools

import jax
import jax.numpy as jnp
from jax import lax
from jax.experimental import pallas as pl
from jax.experimental.pallas import tpu as pltpu
from jax.experimental.pallas import tpu_sc as plsc

B, D, HW = 32, 32, 64 * 64
E = 512
_LOSS_SCALE = 1.25 / (B * HW * D)


def _tc_body(cb_ref, z_ref, idx_ref, loss_ref):
    b = pl.program_id(0)
    zb = z_ref[0].reshape(D, HW)
    cb = cb_ref[...]

    nscores2 = lax.dot_general(-2.0 * cb, zb, (((1,), (0,)), ((), ())),
                               preferred_element_type=jnp.float32)
    e_sq = jnp.sum(cb * cb, axis=1, keepdims=True)
    z_sq = jnp.sum(zb * zb, axis=0, keepdims=True)
    t = z_sq + e_sq
    dist = t + nscores2

    m = jnp.min(dist, axis=0, keepdims=True)
    eidx = lax.broadcasted_iota(jnp.int32, (E, 1), 0).astype(jnp.float32)
    idx_f = jnp.min(jnp.where(dist == m, eidx, float(E)), axis=0)
    idx_ref[0, 0] = idx_f.astype(jnp.int32)

    @pl.when(b == 0)
    def _init():
        loss_ref[0, 0] = 0.0

    loss_ref[0, 0] += jnp.sum(m)


def _tc_stage(codebook, z4, b0, nb):
    return pl.pallas_call(
        _tc_body,
        grid=(nb,),
        in_specs=[
            pl.BlockSpec((E, D), lambda b: (0, 0)),
            pl.BlockSpec((1, D, 64, 64), lambda b: (b0 + b, 0, 0, 0)),
        ],
        out_specs=[
            pl.BlockSpec((1, 1, HW), lambda b: (b, 0, 0)),
            pl.BlockSpec(memory_space=pltpu.SMEM),
        ],
        out_shape=[
            jax.ShapeDtypeStruct((nb, 1, HW), jnp.int32),
            jax.ShapeDtypeStruct((1, 1), jnp.float32),
        ],
    )(codebook, z4)


_CHUNK = 1024
_NCHUNK = HW // _CHUNK


def _sc_gather(cbt, idx3, nb):
    mesh = plsc.VectorSubcoreMesh(core_axis_name="c", subcore_axis_name="s")
    wpb = 32 // nb
    per_w = HW // wpb
    nchunk = per_w // _CHUNK

    @functools.partial(
        pl.kernel,
        out_type=jax.ShapeDtypeStruct((nb, D, HW), jnp.float32),
        mesh=mesh,
        compiler_params=pltpu.CompilerParams(
            use_tc_tiling_on_sc=False, needs_layout_passes=False),
        scratch_types=[
            pltpu.VMEM((D * E,), jnp.float32),
            pltpu.VMEM((per_w,), jnp.int32),
            pltpu.VMEM((D, _CHUNK), jnp.float32),
            pltpu.VMEM((D, _CHUNK), jnp.float32),
            pltpu.SemaphoreType.DMA,
            pltpu.SemaphoreType.DMA,
        ],
    )
    def body(cbt_hbm, idx_hbm, zq_hbm, cbt_v, idx_v, out0, out1, sem0, sem1):
        w = lax.axis_index("s") * 2 + lax.axis_index("c")
        wb = w // wpb
        off = (w % wpb) * per_w
        bufs, sems = (out0, out1), (sem0, sem1)
        pltpu.sync_copy(cbt_hbm, cbt_v)
        pltpu.sync_copy(idx_hbm.at[wb, 0, pl.ds(off, per_w)], idx_v)
        for c in range(nchunk):
            buf, sem = bufs[c % 2], sems[c % 2]
            if c >= 2:
                pltpu.make_async_copy(
                    buf, zq_hbm.at[wb, :, pl.ds(off + (c - 2) * _CHUNK,
                                                _CHUNK)], sem).wait()

            @plsc.parallel_loop(0, _CHUNK // 16, 1, unroll=4)
            def g_body(g, _c=c, _buf=buf):
                iv = idx_v[pl.ds(_c * _CHUNK + g * 16, 16)]
                for d in range(D):
                    _buf[d, pl.ds(g * 16, 16)] = plsc.load_gather(
                        cbt_v, [iv + (d * E)])

            pltpu.async_copy(
                buf, zq_hbm.at[wb, :, pl.ds(off + c * _CHUNK, _CHUNK)], sem)
        for c in (nchunk - 2, nchunk - 1):
            pltpu.make_async_copy(
                bufs[c % 2],
                zq_hbm.at[wb, :, pl.ds(off + c * _CHUNK, _CHUNK)],
                sems[c % 2]).wait()

    return body(cbt, idx3)


def kernel(z_e, codebook):
    idx3, loss = _tc_stage(codebook, z_e, 0, B)
    zq3 = _sc_gather(codebook.T.reshape(-1), idx3, B)
    z_q = zq3.reshape(B, D, 64, 64)
    indices = idx3.reshape(B, 64, 64)
    return (z_q, indices, loss[0, 0] * _LOSS_SCALE)


# speedup vs baseline: 1.1356x; 1.0156x over previous
import functools

import jax
import jax.numpy as jnp
from jax import lax
from jax.experimental import pallas as pl
from jax.experimental.pallas import tpu as pltpu
from jax.experimental.pallas import tpu_sc as plsc

B, D, HW = 32, 32, 64 * 64
E = 512
_LOSS_SCALE = 1.25 / (B * HW * D)


def _tc_body(cb_ref, z_ref, idx_ref, loss_ref):
    b = pl.program_id(0)
    zb = z_ref[0].reshape(D, HW)
    cb = cb_ref[...]

    nscores2 = lax.dot_general(-2.0 * cb, zb, (((1,), (0,)), ((), ())),
                               preferred_element_type=jnp.float32)
    e_sq = jnp.sum(cb * cb, axis=1, keepdims=True)
    z_sq = jnp.sum(zb * zb, axis=0, keepdims=True)
    t = z_sq + e_sq
    dist = t + nscores2

    m = jnp.min(dist, axis=0, keepdims=True)
    eidx = lax.broadcasted_iota(jnp.int32, (E, 1), 0).astype(jnp.float32)
    idx_f = jnp.min(jnp.where(dist == m, eidx, float(E)), axis=0)
    idx_ref[0, 0] = idx_f.astype(jnp.int32)

    @pl.when(b == 0)
    def _init():
        loss_ref[0, 0] = 0.0

    loss_ref[0, 0] += jnp.sum(m)


def _tc_stage(codebook, z4, b0, nb):
    return pl.pallas_call(
        _tc_body,
        grid=(nb,),
        in_specs=[
            pl.BlockSpec((E, D), lambda b: (0, 0)),
            pl.BlockSpec((1, D, 64, 64), lambda b: (b0 + b, 0, 0, 0)),
        ],
        out_specs=[
            pl.BlockSpec((1, 1, HW), lambda b: (b, 0, 0)),
            pl.BlockSpec(memory_space=pltpu.SMEM),
        ],
        out_shape=[
            jax.ShapeDtypeStruct((nb, 1, HW), jnp.int32),
            jax.ShapeDtypeStruct((1, 1), jnp.float32),
        ],
    )(codebook, z4)


_CHUNK = 1024
_NCHUNK = HW // _CHUNK


def _sc_gather(cbt, idx3, nb):
    mesh = plsc.VectorSubcoreMesh(core_axis_name="c", subcore_axis_name="s")
    wpb = 32 // nb
    per_w = HW // wpb
    nchunk = per_w // _CHUNK

    @functools.partial(
        pl.kernel,
        out_type=jax.ShapeDtypeStruct((nb, D, HW), jnp.float32),
        mesh=mesh,
        compiler_params=pltpu.CompilerParams(
            use_tc_tiling_on_sc=False, needs_layout_passes=False),
        scratch_types=[
            pltpu.VMEM((D * E,), jnp.float32),
            pltpu.VMEM((per_w,), jnp.int32),
            pltpu.VMEM((D, _CHUNK), jnp.float32),
            pltpu.VMEM((D, _CHUNK), jnp.float32),
            pltpu.SemaphoreType.DMA,
            pltpu.SemaphoreType.DMA,
        ],
    )
    def body(cbt_hbm, idx_hbm, zq_hbm, cbt_v, idx_v, out0, out1, sem0, sem1):
        w = lax.axis_index("s") * 2 + lax.axis_index("c")
        wb = w // wpb
        off = (w % wpb) * per_w
        bufs, sems = (out0, out1), (sem0, sem1)
        pltpu.sync_copy(cbt_hbm, cbt_v)
        pltpu.sync_copy(idx_hbm.at[wb, 0, pl.ds(off, per_w)], idx_v)
        for c in range(nchunk):
            buf, sem = bufs[c % 2], sems[c % 2]
            if c >= 2:
                pltpu.make_async_copy(
                    buf, zq_hbm.at[wb, :, pl.ds(off + (c - 2) * _CHUNK,
                                                _CHUNK)], sem).wait()

            @plsc.parallel_loop(0, _CHUNK // 16, 1, unroll=8)
            def g_body(g, _c=c, _buf=buf):
                iv = idx_v[pl.ds(_c * _CHUNK + g * 16, 16)]
                for d in range(D):
                    _buf[d, pl.ds(g * 16, 16)] = plsc.load_gather(
                        cbt_v, [iv + (d * E)])

            pltpu.async_copy(
                buf, zq_hbm.at[wb, :, pl.ds(off + c * _CHUNK, _CHUNK)], sem)
        for c in (nchunk - 2, nchunk - 1):
            pltpu.make_async_copy(
                bufs[c % 2],
                zq_hbm.at[wb, :, pl.ds(off + c * _CHUNK, _CHUNK)],
                sems[c % 2]).wait()

    return body(cbt, idx3)


def kernel(z_e, codebook):
    idx3, loss = _tc_stage(codebook, z_e, 0, B)
    zq3 = _sc_gather(codebook.T.reshape(-1), idx3, B)
    z_q = zq3.reshape(B, D, 64, 64)
    indices = idx3.reshape(B, 64, 64)
    return (z_q, indices, loss[0, 0] * _LOSS_SCALE)
